# baseline (speedup 1.0000x reference)
import functools

import jax
import jax.numpy as jnp
from jax import lax
from jax.experimental import pallas as pl
from jax.experimental.pallas import tpu as pltpu
from jax.experimental.pallas import tpu_sc as plsc

DIM = 128
B = 4096
L = 20
NW = 32
BW = B // NW
KW = BW * L
CHUNK = 128
NCHUNK = KW // CHUNK
SENTINEL = -1e30

_MESH = plsc.VectorSubcoreMesh(core_axis_name="c", subcore_axis_name="s")

_GATHER_DNUMS = lax.GatherDimensionNumbers(
    offset_dims=(), collapsed_slice_dims=(0,), start_index_map=(0,))


def _shfl(v, idx):
    return lax.gather(v, idx[:, None], _GATHER_DNUMS, slice_sizes=(1,),
                      mode=lax.GatherScatterMode.PROMISE_IN_BOUNDS)


@functools.partial(
    pl.kernel,
    out_type=jax.ShapeDtypeStruct((NW, KW), jnp.float32),
    mesh=_MESH,
    scratch_types=[
        pltpu.VMEM((NCHUNK, CHUNK), jnp.int32),
        pltpu.VMEM((4 * CHUNK, DIM), jnp.float32),
        pltpu.VMEM((BW, DIM), jnp.float32),
        pltpu.VMEM((BW, L), jnp.int32),
        pltpu.VMEM((BW + 16,), jnp.int32),
        pltpu.VMEM((KW,), jnp.float32),
        pltpu.SemaphoreType.DMA,
        pltpu.SemaphoreType.DMA,
        pltpu.SemaphoreType.DMA,
        pltpu.SemaphoreType.DMA,
    ],
)
def _sc_logits(table_hbm, tp_hbm, hid_hbm, code_hbm, len_hbm, out_hbm,
               idx_v, rows_v, hid_v, code_v, len_v, z_v,
               sem0, sem1, sem2, sem3):
    wid = lax.axis_index("c") * 16 + lax.axis_index("s")
    base = wid * BW
    pltpu.sync_copy(tp_hbm.at[wid], idx_v)
    pltpu.sync_copy(hid_hbm.at[pl.ds(base, BW)], hid_v)
    pltpu.sync_copy(code_hbm.at[pl.ds(base, BW)], code_v)
    pltpu.sync_copy(len_hbm.at[pl.ds(base, BW)], len_v.at[pl.ds(0, BW)])
    lanes = lax.iota(jnp.int32, 16)

    bufs = [rows_v.at[pl.ds(p * CHUNK, CHUNK)] for p in range(4)]
    sems = [sem0, sem1, sem2, sem3]

    def start_gather(c, buf, sem):
        cc = jnp.minimum(c, NCHUNK - 1)
        pltpu.async_copy(table_hbm.at[idx_v.at[cc]], buf, sem)

    def wait_gather(buf, sem):
        pltpu.make_async_copy(table_hbm.at[idx_v.at[0]], buf, sem).wait()

    def compute(c, off):
        def blk_body(t, _):
            k0 = c * CHUNK + t * 16
            b0 = k0 // L
            b1 = (k0 + 15) // L
            l0 = k0 - b0 * L
            ub = b1 * L - k0
            h0 = [hid_v[b0, pl.ds(s * 16, 16)] for s in range(DIM // 16)]
            h1 = [hid_v[b1, pl.ds(s * 16, 16)] for s in range(DIM // 16)]
            zvec = jnp.zeros((16,), jnp.float32)
            for u in range(16):
                r = t * 16 + u
                in_b0 = u < ub
                acc = None
                for s in range(DIM // 16):
                    h = jnp.where(in_b0, h0[s], h1[s])
                    prod = rows_v[off + r, pl.ds(s * 16, 16)] * h
                    acc = prod if acc is None else acc + prod
                for sh in (8, 4, 2, 1):
                    acc = acc + _shfl(acc, jnp.bitwise_xor(lanes, sh))
                zvec = jnp.where(lanes == u, acc, zvec)
            lvec = l0 + lanes
            lvec = jnp.where(lvec >= L, lvec - L, lvec)
            len0 = len_v[pl.ds(b0, 16)][0]
            len1 = len_v[pl.ds(b1, 16)][0]
            lenv = jnp.where(lanes < ub, len0, len1)
            st0 = jnp.minimum(l0, L - 16)
            code0 = _shfl(code_v[b0, pl.ds(st0, 16)],
                          jnp.minimum(lanes + (l0 - st0), 15))
            code1 = _shfl(code_v[b1, pl.ds(0, 16)],
                          jnp.maximum(lanes - ub, 0))
            codev = jnp.where(lanes < ub, code0, code1)
            sign = 1.0 - 2.0 * codev.astype(jnp.float32)
            x = jnp.where(lvec < lenv, zvec * sign, jnp.float32(SENTINEL))
            z_v[pl.ds(k0, 16)] = x
            return 0

        lax.fori_loop(0, CHUNK // 16, blk_body, 0)

    start_gather(0, bufs[0], sem0)
    start_gather(1, bufs[1], sem1)
    start_gather(2, bufs[2], sem2)

    def chunk_body(c, _):
        ph = c % 4
        for p in range(4):
            @pl.when(ph == p)
            def _(p=p):
                wait_gather(bufs[p], sems[p])
                start_gather(c + 3, bufs[(p + 3) % 4], sems[(p + 3) % 4])

        compute(c, ph * CHUNK)
        return 0

    lax.fori_loop(0, NCHUNK, chunk_body, 0)
    wait_gather(bufs[0], sems[0])
    wait_gather(bufs[1], sems[1])
    wait_gather(bufs[2], sems[2])
    pltpu.sync_copy(z_v, out_hbm.at[wid])


def _tc_loss_body(x_ref, out_ref):
    x = x_ref[...]
    loss = jnp.maximum(x, 0.0) + jnp.log(1.0 + jnp.exp(-jnp.abs(x)))
    cnt = jnp.sum((x > SENTINEL * 0.5).astype(jnp.float32))
    out_ref[...] = (jnp.sum(loss) / cnt).reshape(1, 1)


_tc_loss = pl.pallas_call(
    _tc_loss_body,
    out_shape=jax.ShapeDtypeStruct((1, 1), jnp.float32),
)


def kernel(hidden_, target, target_path, target_path_len, target_code, embed_table):
    tp = target_path.reshape(NW, NCHUNK, CHUNK)
    x = _sc_logits(embed_table, tp, hidden_, target_code, target_path_len)
    loss = _tc_loss(x)
    return loss[0, 0]


# speedup vs baseline: 1.0160x; 1.0160x over previous
import functools

import jax
import jax.numpy as jnp
from jax import lax
from jax.experimental import pallas as pl
from jax.experimental.pallas import tpu as pltpu
from jax.experimental.pallas import tpu_sc as plsc

DIM = 128
B = 4096
L = 20
NW = 32
BW = B // NW
KW = BW * L
CHUNK = 128
NCHUNK = KW // CHUNK
SENTINEL = -1e30

_MESH = plsc.VectorSubcoreMesh(core_axis_name="c", subcore_axis_name="s")

_GATHER_DNUMS = lax.GatherDimensionNumbers(
    offset_dims=(), collapsed_slice_dims=(0,), start_index_map=(0,))


def _shfl(v, idx):
    return lax.gather(v, idx[:, None], _GATHER_DNUMS, slice_sizes=(1,),
                      mode=lax.GatherScatterMode.PROMISE_IN_BOUNDS)


@functools.partial(
    pl.kernel,
    out_type=jax.ShapeDtypeStruct((NW, KW), jnp.float32),
    mesh=_MESH,
    scratch_types=[
        pltpu.VMEM((NCHUNK, CHUNK), jnp.int32),
        pltpu.VMEM((4 * CHUNK, DIM), jnp.float32),
        pltpu.VMEM((BW, DIM), jnp.float32),
        pltpu.VMEM((BW, L), jnp.int32),
        pltpu.VMEM((BW + 16,), jnp.int32),
        pltpu.VMEM((KW,), jnp.float32),
        pltpu.SemaphoreType.DMA,
        pltpu.SemaphoreType.DMA,
        pltpu.SemaphoreType.DMA,
        pltpu.SemaphoreType.DMA,
    ],
)
def _sc_logits(table_hbm, tp_hbm, hid_hbm, code_hbm, len_hbm, out_hbm,
               idx_v, rows_v, hid_v, code_v, len_v, z_v,
               sem0, sem1, sem2, sem3):
    wid = lax.axis_index("c") * 16 + lax.axis_index("s")
    base = wid * BW
    pltpu.sync_copy(tp_hbm.at[wid], idx_v)
    pltpu.sync_copy(hid_hbm.at[pl.ds(base, BW)], hid_v)
    pltpu.sync_copy(code_hbm.at[pl.ds(base, BW)], code_v)
    pltpu.sync_copy(len_hbm.at[pl.ds(base, BW)], len_v.at[pl.ds(0, BW)])
    lanes = lax.iota(jnp.int32, 16)

    bufs = [rows_v.at[pl.ds(p * CHUNK, CHUNK)] for p in range(4)]
    sems = [sem0, sem1, sem2, sem3]

    def start_gather(c, buf, sem):
        cc = jnp.minimum(c, NCHUNK - 1)
        pltpu.async_copy(table_hbm.at[idx_v.at[cc]], buf, sem)

    def wait_gather(buf, sem):
        pltpu.make_async_copy(table_hbm.at[idx_v.at[0]], buf, sem).wait()

    def compute(c, off):
        def blk_body(t, _):
            k0 = c * CHUNK + t * 16
            b0 = k0 // L
            b1 = (k0 + 15) // L
            l0 = k0 - b0 * L
            ub = b1 * L - k0
            h0 = [hid_v[b0, pl.ds(s * 16, 16)] for s in range(DIM // 16)]
            h1 = [hid_v[b1, pl.ds(s * 16, 16)] for s in range(DIM // 16)]
            zvec = jnp.zeros((16,), jnp.float32)
            for u in range(16):
                r = t * 16 + u
                in_b0 = u < ub
                acc = None
                for s in range(DIM // 16):
                    h = jnp.where(in_b0, h0[s], h1[s])
                    prod = rows_v[off + r, pl.ds(s * 16, 16)] * h
                    acc = prod if acc is None else acc + prod
                for sh in (8, 4, 2, 1):
                    acc = acc + _shfl(acc, jnp.bitwise_xor(lanes, sh))
                zvec = jnp.where(lanes == u, acc, zvec)
            lvec = l0 + lanes
            lvec = jnp.where(lvec >= L, lvec - L, lvec)
            len0 = len_v[pl.ds(b0, 16)][0]
            len1 = len_v[pl.ds(b1, 16)][0]
            lenv = jnp.where(lanes < ub, len0, len1)
            st0 = jnp.minimum(l0, L - 16)
            code0 = _shfl(code_v[b0, pl.ds(st0, 16)],
                          jnp.minimum(lanes + (l0 - st0), 15))
            code1 = _shfl(code_v[b1, pl.ds(0, 16)],
                          jnp.maximum(lanes - ub, 0))
            codev = jnp.where(lanes < ub, code0, code1)
            sign = 1.0 - 2.0 * codev.astype(jnp.float32)
            x = jnp.where(lvec < lenv, zvec * sign, jnp.float32(SENTINEL))
            z_v[pl.ds(k0, 16)] = x
            return 0

        lax.fori_loop(0, CHUNK // 16, blk_body, 0)

    start_gather(0, bufs[0], sem0)
    start_gather(1, bufs[1], sem1)

    def chunk_body(c, _):
        ph = c % 4
        for p in range(4):
            @pl.when(ph == p)
            def _(p=p):
                wait_gather(bufs[p], sems[p])
                start_gather(c + 2, bufs[(p + 2) % 4], sems[(p + 2) % 4])

        compute(c, ph * CHUNK)
        return 0

    lax.fori_loop(0, NCHUNK, chunk_body, 0)
    wait_gather(bufs[0], sems[0])
    wait_gather(bufs[1], sems[1])
    pltpu.sync_copy(z_v, out_hbm.at[wid])


def _tc_loss_body(x_ref, out_ref):
    x = x_ref[...]
    loss = jnp.maximum(x, 0.0) + jnp.log(1.0 + jnp.exp(-jnp.abs(x)))
    cnt = jnp.sum((x > SENTINEL * 0.5).astype(jnp.float32))
    out_ref[...] = (jnp.sum(loss) / cnt).reshape(1, 1)


_tc_loss = pl.pallas_call(
    _tc_loss_body,
    out_shape=jax.ShapeDtypeStruct((1, 1), jnp.float32),
)


def kernel(hidden_, target, target_path, target_path_len, target_code, embed_table):
    tp = target_path.reshape(NW, NCHUNK, CHUNK)
    x = _sc_logits(embed_table, tp, hidden_, target_code, target_path_len)
    loss = _tc_loss(x)
    return loss[0, 0]
